# baseline (speedup 1.0000x reference)
import numpy as np
import jax
import jax.numpy as jnp
from jax.experimental import pallas as pl
from jax.experimental.pallas import tpu as pltpu

_N, _INN, _INC, _OUTC, _OUTN = 128, 512, 256, 256, 256
_BB = 32


def _pool_matrix():
    o = np.arange(_OUTN)[:, None]
    i = np.arange(_INN)[None, :]
    p = (i == 2 * o) | ((i == 2 * o + 1) & (o % 2 == 0))
    return p.astype(np.float32)


def _fgl_body(x_ref, p_ref, w_ref, b_ref, o_ref):
    p = p_ref[...]
    w = w_ref[...]
    b = b_ref[...]
    for bb in range(_BB):
        pooled = jnp.dot(p, x_ref[bb], preferred_element_type=jnp.float32)
        o_ref[bb] = jnp.dot(pooled, w, preferred_element_type=jnp.float32) + b


def kernel(x, W, bias):
    P = jnp.asarray(_pool_matrix())
    grid = (_N // _BB,)
    return pl.pallas_call(
        _fgl_body,
        grid=grid,
        in_specs=[
            pl.BlockSpec((_BB, _INN, _INC), lambda i: (i, 0, 0)),
            pl.BlockSpec((_OUTN, _INN), lambda i: (0, 0)),
            pl.BlockSpec((_INC, _OUTC), lambda i: (0, 0)),
            pl.BlockSpec((_OUTN, _OUTC), lambda i: (0, 0)),
        ],
        out_specs=pl.BlockSpec((_BB, _OUTN, _OUTC), lambda i: (i, 0, 0)),
        out_shape=jax.ShapeDtypeStruct((_N, _OUTN, _OUTC), jnp.float32),
    )(x, P, W, bias)


# speedup vs baseline: 1.0352x; 1.0352x over previous
import numpy as np
import jax
import jax.numpy as jnp
from jax.experimental import pallas as pl
from jax.experimental.pallas import tpu as pltpu

_N, _INN, _INC, _OUTC, _OUTN = 128, 512, 256, 256, 256
_BB = 16


def _pool_matrix():
    o = np.arange(_OUTN)[:, None]
    i = np.arange(_INN)[None, :]
    p = (i == 2 * o) | ((i == 2 * o + 1) & (o % 2 == 0))
    return p.astype(np.float32)


def _fgl_body(x_ref, p_ref, w_ref, b_ref, o_ref):
    p = p_ref[...]
    w = w_ref[...]
    b = b_ref[...]
    for bb in range(_BB):
        xb = x_ref[bb].astype(jnp.bfloat16)
        pooled = jnp.dot(p, xb, preferred_element_type=jnp.float32)
        o_ref[bb] = jnp.dot(pooled, w, preferred_element_type=jnp.float32) + b


def kernel(x, W, bias):
    P = jnp.asarray(_pool_matrix()).astype(jnp.bfloat16)
    grid = (_N // _BB,)
    return pl.pallas_call(
        _fgl_body,
        grid=grid,
        in_specs=[
            pl.BlockSpec((_BB, _INN, _INC), lambda i: (i, 0, 0)),
            pl.BlockSpec((_OUTN, _INN), lambda i: (0, 0)),
            pl.BlockSpec((_INC, _OUTC), lambda i: (0, 0)),
            pl.BlockSpec((_OUTN, _OUTC), lambda i: (0, 0)),
        ],
        out_specs=pl.BlockSpec((_BB, _OUTN, _OUTC), lambda i: (i, 0, 0)),
        out_shape=jax.ShapeDtypeStruct((_N, _OUTN, _OUTC), jnp.float32),
    )(x, P, W, bias)
